# Initial kernel scaffold; baseline (speedup 1.0000x reference)
import jax
import jax.numpy as jnp
from jax.experimental import pallas as pl


def kernel(xyz, params):
    raise NotImplementedError("write your pallas kernel here")


# speedup vs baseline: 5.3294x; 5.3294x over previous
import functools

import jax
import jax.numpy as jnp
from jax import lax
from jax.experimental import pallas as pl

F32 = jnp.float32
HI = lax.Precision.HIGHEST
_INTERPRET = False


def _dot(a, b):
    return lax.dot_general(a, b, (((1,), (0,)), ((), ())),
                           precision=HI, preferred_element_type=F32)


def _fps_body(npoint, n, xt_ref, out_ref):
    x = xt_ref[:, 0, :]
    y = xt_ref[:, 1, :]
    z = xt_ref[:, 2, :]
    bb = x.shape[0]
    lanes = lax.broadcasted_iota(jnp.int32, (bb, n), 1)
    slot = lax.broadcasted_iota(jnp.int32, (bb, npoint), 1)

    def body(i, carry):
        dist, far, cx, cy, cz = carry
        m = lanes == far
        c0 = jnp.sum(jnp.where(m, x, 0.0), axis=1, keepdims=True)
        c1 = jnp.sum(jnp.where(m, y, 0.0), axis=1, keepdims=True)
        c2 = jnp.sum(jnp.where(m, z, 0.0), axis=1, keepdims=True)
        hit = slot == i
        cx = jnp.where(hit, c0, cx)
        cy = jnp.where(hit, c1, cy)
        cz = jnp.where(hit, c2, cz)
        dx = x - c0
        dy = y - c1
        dz = z - c2
        dn = (dx * dx + dy * dy) + dz * dz
        dist = jnp.minimum(dist, dn)
        mv = jnp.max(dist, axis=1, keepdims=True)
        far = jnp.min(jnp.where(dist == mv, lanes, n), axis=1, keepdims=True)
        return dist, far, cx, cy, cz

    zc = jnp.zeros((bb, npoint), F32)
    _, _, cx, cy, cz = lax.fori_loop(
        0, npoint, body,
        (jnp.full((bb, n), 1e10, F32), jnp.zeros((bb, 1), jnp.int32),
         zc, zc, zc))
    out_ref[:, 0:1, :] = cx[:, None, :]
    out_ref[:, 1:2, :] = cy[:, None, :]
    out_ref[:, 2:3, :] = cz[:, None, :]


def _fps(xt, npoint):
    bb, _, n = xt.shape
    return pl.pallas_call(
        functools.partial(_fps_body, npoint, n),
        out_shape=jax.ShapeDtypeStruct((bb, 3, npoint), F32),
        interpret=_INTERPRET,
    )(xt)


def _cumsum_lanes(c, n):
    sh = 1
    while sh < n:
        pad = jnp.zeros((c.shape[0], sh), jnp.int32)
        c = c + jnp.concatenate([pad, c[:, :n - sh]], axis=1)
        sh *= 2
    return c


def _group_body(r2, kk, n, sb, cf, feat_ref, ptsT_ref, newp_ref, out_ref):
    ptsT = ptsT_ref[0]
    newp = newp_ref[0]
    rn = (ptsT[0:1] * ptsT[0:1] + ptsT[1:2] * ptsT[1:2]) + ptsT[2:3] * ptsT[2:3]
    s0 = newp[:, 0:1]
    s1 = newp[:, 1:2]
    s2 = newp[:, 2:3]
    sn = (s0 * s0 + s1 * s1) + s2 * s2
    d = (-2.0 * _dot(newp, ptsT) + sn) + rn
    mask = (d <= r2).astype(jnp.int32)
    c = _cumsum_lanes(mask, n)
    total = c[:, n - 1:n]
    cm = c * mask
    kv = lax.broadcasted_iota(jnp.int32, (1, kk), 1) + 1
    target = jnp.where(kv <= total, kv, 1)
    oh = (cm[:, None, :] == target[:, :, None]).reshape(sb * kk, n).astype(F32)
    g = _dot(oh, feat_ref[0])
    ctr = jnp.broadcast_to(newp[:, None, :], (sb, kk, 3)).reshape(sb * kk, 3)
    if cf > 3:
        out_ref[0, :, :cf - 3] = g[:, :cf - 3]
    out_ref[0, :, cf - 3:] = g[:, cf - 3:] - ctr


def _group(feat, ptsT, newp, radius, kk, sbb):
    bb, n, cf = feat.shape
    s = newp.shape[1]
    return pl.pallas_call(
        functools.partial(_group_body, radius * radius, kk, n, sbb, cf),
        grid=(bb, s // sbb),
        in_specs=[
            pl.BlockSpec((1, n, cf), lambda b, i: (b, 0, 0)),
            pl.BlockSpec((1, 3, n), lambda b, i: (b, 0, 0)),
            pl.BlockSpec((1, sbb, 3), lambda b, i: (b, i, 0)),
        ],
        out_specs=pl.BlockSpec((1, sbb * kk, cf), lambda b, i: (b, i, 0)),
        out_shape=jax.ShapeDtypeStruct((bb, s * kk, cf), F32),
        interpret=_INTERPRET,
    )(feat, ptsT, newp)


def _norm_relu(x, s, gb, rr):
    mean = s[0:1] * (1.0 / rr)
    ex2 = s[1:2] * (1.0 / rr)
    var = ex2 - mean * mean
    inv = 1.0 / jnp.sqrt(var + 1e-5)
    return jnp.maximum((x - mean) * inv * gb[0:1] + gb[1:2], 0.0)


def _layer_body(rr, first, *refs):
    if first:
        x_ref, wt_ref, b_ref, out_ref, acc_ref = refs
        x = x_ref[...]
    else:
        x_ref, wt_ref, b_ref, stats_ref, gb_ref, out_ref, acc_ref = refs
        x = _norm_relu(x_ref[...], stats_ref[...], gb_ref[...], rr)
    y = _dot(x, wt_ref[...]) + b_ref[...]
    out_ref[...] = y

    @pl.when(pl.program_id(0) == 0)
    def _():
        acc_ref[...] = jnp.zeros_like(acc_ref)

    s1 = jnp.sum(y, axis=0, keepdims=True)
    s2 = jnp.sum(y * y, axis=0, keepdims=True)
    acc_ref[...] += jnp.concatenate([s1, s2], axis=0)


def _layer(x, stats, gb, wt, b, rb):
    rr, cin = x.shape
    cout = wt.shape[1]
    first = stats is None
    grid = (rr // rb,)
    in_specs = [
        pl.BlockSpec((rb, cin), lambda i: (i, 0)),
        pl.BlockSpec((cin, cout), lambda i: (0, 0)),
        pl.BlockSpec((1, cout), lambda i: (0, 0)),
    ]
    args = [x, wt, b]
    if not first:
        in_specs += [pl.BlockSpec((2, cin), lambda i: (0, 0)),
                     pl.BlockSpec((2, cin), lambda i: (0, 0))]
        args += [stats, gb]
    y, acc = pl.pallas_call(
        functools.partial(_layer_body, float(rr), first),
        grid=grid,
        in_specs=in_specs,
        out_specs=[pl.BlockSpec((rb, cout), lambda i: (i, 0)),
                   pl.BlockSpec((2, cout), lambda i: (0, 0))],
        out_shape=[jax.ShapeDtypeStruct((rr, cout), F32),
                   jax.ShapeDtypeStruct((2, cout), F32)],
        interpret=_INTERPRET,
    )(*args)
    return y, acc


def _finish_body(rr, kk, rb, cc, x_ref, stats_ref, gb_ref, out_ref):
    x = x_ref[...]
    if kk > 1:
        x = jnp.max(x.reshape(rb // kk, kk, cc), axis=1)
    out_ref[...] = _norm_relu(x, stats_ref[...], gb_ref[...], rr)


def _finish(x, stats, gb, kk, rb):
    rr, cc = x.shape
    return pl.pallas_call(
        functools.partial(_finish_body, float(rr), kk, rb, cc),
        grid=(rr // rb,),
        in_specs=[
            pl.BlockSpec((rb, cc), lambda i: (i, 0)),
            pl.BlockSpec((2, cc), lambda i: (0, 0)),
            pl.BlockSpec((2, cc), lambda i: (0, 0)),
        ],
        out_specs=pl.BlockSpec((rb // kk, cc), lambda i: (i, 0)),
        out_shape=jax.ShapeDtypeStruct((rr // kk, cc), F32),
        interpret=_INTERPRET,
    )(x, stats, gb)


def _mlp_rows(x, layers, kk):
    rr = x.shape[0]
    rb = min(rr, 2048)
    stats = None
    gb = None
    for lyr in layers:
        wt = jnp.transpose(lyr['W'], (1, 0))
        b = lyr['b'][None, :]
        x, stats = _layer(x, stats, gb, wt, b, rb)
        gb = jnp.stack([lyr['gamma'], lyr['beta']])
    fin_rb = rb if rb % kk == 0 and rb >= kk else rr
    return _finish(x, stats, gb, kk, fin_rb)


def _interp_body(s2, c1, nb, x1_ref, x2T_ref, p2_ref, *rest):
    if c1 > 0:
        p1_ref, out_ref = rest
    else:
        (out_ref,) = rest
    x1 = x1_ref[0]
    x2T = x2T_ref[0]
    n1 = (x1[:, 0:1] * x1[:, 0:1] + x1[:, 1:2] * x1[:, 1:2]) + x1[:, 2:3] * x1[:, 2:3]
    n2 = (x2T[0:1] * x2T[0:1] + x2T[1:2] * x2T[1:2]) + x2T[2:3] * x2T[2:3]
    d = (-2.0 * _dot(x1, x2T) + n1) + n2
    lanes = lax.broadcasted_iota(jnp.int32, (nb, s2), 1)
    wmat = jnp.zeros((nb, s2), F32)
    wsum = jnp.zeros((nb, 1), F32)
    for _ in range(3):
        mv = jnp.min(d, axis=1, keepdims=True)
        idx = jnp.min(jnp.where(d == mv, lanes, s2), axis=1, keepdims=True)
        oh = lanes == idx
        w = 1.0 / (mv + 1e-8)
        wmat = wmat + jnp.where(oh, w, 0.0)
        wsum = wsum + w
        d = jnp.where(oh, 1e30, d)
    wmat = wmat / wsum
    interp = _dot(wmat, p2_ref[0])
    if c1 > 0:
        out_ref[0, :, :c1] = p1_ref[0]
        out_ref[0, :, c1:] = interp
    else:
        out_ref[0] = interp


def _interp(x1p, x2T, p2, p1, nb):
    bb, n1, _ = x1p.shape
    s2 = x2T.shape[2]
    c2 = p2.shape[2]
    c1 = 0 if p1 is None else p1.shape[2]
    in_specs = [
        pl.BlockSpec((1, nb, 3), lambda b, i: (b, i, 0)),
        pl.BlockSpec((1, 3, s2), lambda b, i: (b, 0, 0)),
        pl.BlockSpec((1, s2, c2), lambda b, i: (b, 0, 0)),
    ]
    args = [x1p, x2T, p2]
    if c1 > 0:
        in_specs.append(pl.BlockSpec((1, nb, c1), lambda b, i: (b, i, 0)))
        args.append(p1)
    return pl.pallas_call(
        functools.partial(_interp_body, s2, c1, nb),
        grid=(bb, n1 // nb),
        in_specs=in_specs,
        out_specs=pl.BlockSpec((1, nb, c1 + c2), lambda b, i: (b, i, 0)),
        out_shape=jax.ShapeDtypeStruct((bb, n1, c1 + c2), F32),
        interpret=_INTERPRET,
    )(*args)


SA_LEVELS = [
    (1024, [0.05, 0.1], [16, 32], 32),
    (256, [0.1, 0.2], [16, 32], 64),
    (64, [0.2, 0.4], [16, 32], 64),
]


def kernel(xyz, params):
    bb = xyz.shape[0]
    l0T = xyz
    l0p = jnp.transpose(xyz, (0, 2, 1))

    ptsT, pts, points = l0T, l0p, None
    level_data = []
    for li, (s, radii, ks, sbb) in enumerate(SA_LEVELS):
        name = 'sa%d' % (li + 1)
        newT = _fps(ptsT, s)
        newp = jnp.transpose(newT, (0, 2, 1))
        if points is None:
            feat = pts
        else:
            feat = jnp.concatenate([points, pts], axis=2)
        outs = []
        for radius, kk, layers in zip(radii, ks, params[name]):
            g = _group(feat, ptsT, newp, radius, kk, sbb)
            rows = g.reshape(bb * s * kk, feat.shape[2])
            outs.append(_mlp_rows(rows, layers, kk))
        new_points = jnp.concatenate(outs, axis=1).reshape(bb, s, -1)
        level_data.append((newT, newp, new_points))
        ptsT, pts, points = newT, newp, new_points

    l1T, l1p, l1_points = level_data[0]
    l2T, l2p, l2_points = level_data[1]
    l3T, l3p, l3_points = level_data[2]

    rows4 = jnp.concatenate([l3p.reshape(bb * 64, 3),
                             l3_points.reshape(bb * 64, -1)], axis=1)
    l4rows = _mlp_rows(rows4, params['sa4'], 64)
    global_features = l4rows.reshape(bb, 512)

    interp4 = jnp.broadcast_to(l4rows[:, None, :], (bb, 64, 512))
    np4 = jnp.concatenate([l3_points, interp4], axis=2).reshape(bb * 64, 1024)
    l3f = _mlp_rows(np4, params['fp4'], 1).reshape(bb, 64, 256)

    np3 = _interp(l2p, l3T, l3f, l2_points, 256)
    l2f = _mlp_rows(np3.reshape(bb * 256, 512), params['fp3'], 1).reshape(bb, 256, 256)

    np2 = _interp(l1p, l2T, l2f, l1_points, 512)
    l1f = _mlp_rows(np2.reshape(bb * 1024, 352), params['fp2'], 1).reshape(bb, 1024, 128)

    np1 = _interp(l0p, l1T, l1f, None, 512)
    l0f = _mlp_rows(np1.reshape(bb * 4096, 128), params['fp1'], 1)
    point_features = l0f.reshape(bb, 4096, 128)

    return (global_features, point_features)


# speedup vs baseline: 11.6600x; 2.1879x over previous
import functools

import jax
import jax.numpy as jnp
from jax import lax
from jax.experimental import pallas as pl

F32 = jnp.float32
HI = lax.Precision.HIGHEST
DEF = lax.Precision.DEFAULT
_INTERPRET = False


def _dot(a, b, prec=DEF):
    return lax.dot_general(a, b, (((1,), (0,)), ((), ())),
                           precision=prec, preferred_element_type=F32)


def _fps_body(npoint, n, xt_ref, out_ref):
    x = xt_ref[:, 0, :]
    y = xt_ref[:, 1, :]
    z = xt_ref[:, 2, :]
    bb = x.shape[0]
    lanes = lax.broadcasted_iota(jnp.int32, (bb, n), 1)
    slot = lax.broadcasted_iota(jnp.int32, (bb, npoint), 1)

    def body(i, carry):
        dist, far, cx, cy, cz = carry
        m = lanes == far
        c0 = jnp.sum(jnp.where(m, x, 0.0), axis=1, keepdims=True)
        c1 = jnp.sum(jnp.where(m, y, 0.0), axis=1, keepdims=True)
        c2 = jnp.sum(jnp.where(m, z, 0.0), axis=1, keepdims=True)
        hit = slot == i
        cx = jnp.where(hit, c0, cx)
        cy = jnp.where(hit, c1, cy)
        cz = jnp.where(hit, c2, cz)
        dx = x - c0
        dy = y - c1
        dz = z - c2
        dn = (dx * dx + dy * dy) + dz * dz
        dist = jnp.minimum(dist, dn)
        mv = jnp.max(dist, axis=1, keepdims=True)
        far = jnp.min(jnp.where(dist == mv, lanes, n), axis=1, keepdims=True)
        return dist, far, cx, cy, cz

    zc = jnp.zeros((bb, npoint), F32)
    _, _, cx, cy, cz = lax.fori_loop(
        0, npoint, body,
        (jnp.full((bb, n), 1e10, F32), jnp.zeros((bb, 1), jnp.int32),
         zc, zc, zc))
    out_ref[:, 0:1, :] = cx[:, None, :]
    out_ref[:, 1:2, :] = cy[:, None, :]
    out_ref[:, 2:3, :] = cz[:, None, :]


def _fps(xt, npoint):
    bb, _, n = xt.shape
    return pl.pallas_call(
        functools.partial(_fps_body, npoint, n),
        out_shape=jax.ShapeDtypeStruct((bb, 3, npoint), F32),
        interpret=_INTERPRET,
    )(xt)


def _cumsum_lanes(c, n):
    sh = 1
    while sh < n:
        pad = jnp.zeros((c.shape[0], sh), jnp.int32)
        c = c + jnp.concatenate([pad, c[:, :n - sh]], axis=1)
        sh *= 2
    return c


def _group_body(r2, kk, n, sb, cf, feat_ref, ptsT_ref, newp_ref, out_ref):
    ptsT = ptsT_ref[0]
    newp = newp_ref[0]
    rn = (ptsT[0:1] * ptsT[0:1] + ptsT[1:2] * ptsT[1:2]) + ptsT[2:3] * ptsT[2:3]
    s0 = newp[:, 0:1]
    s1 = newp[:, 1:2]
    s2 = newp[:, 2:3]
    sn = (s0 * s0 + s1 * s1) + s2 * s2
    d = (-2.0 * _dot(newp, ptsT) + sn) + rn
    mask = (d <= r2).astype(jnp.int32)
    c = _cumsum_lanes(mask, n)
    total = c[:, n - 1:n]
    cm = c * mask
    kv = lax.broadcasted_iota(jnp.int32, (1, kk), 1) + 1
    target = jnp.where(kv <= total, kv, 1)
    oh = (cm[:, None, :] == target[:, :, None]).reshape(sb * kk, n).astype(F32)
    g = _dot(oh, feat_ref[0], DEF)
    ctr = jnp.broadcast_to(newp[:, None, :], (sb, kk, 3)).reshape(sb * kk, 3)
    if cf > 3:
        out_ref[0, :, :cf - 3] = g[:, :cf - 3]
    out_ref[0, :, cf - 3:] = g[:, cf - 3:] - ctr


def _group(feat, ptsT, newp, radius, kk, sbb):
    bb, n, cf = feat.shape
    s = newp.shape[1]
    return pl.pallas_call(
        functools.partial(_group_body, radius * radius, kk, n, sbb, cf),
        grid=(bb, s // sbb),
        in_specs=[
            pl.BlockSpec((1, n, cf), lambda b, i: (b, 0, 0)),
            pl.BlockSpec((1, 3, n), lambda b, i: (b, 0, 0)),
            pl.BlockSpec((1, sbb, 3), lambda b, i: (b, i, 0)),
        ],
        out_specs=pl.BlockSpec((1, sbb * kk, cf), lambda b, i: (b, i, 0)),
        out_shape=jax.ShapeDtypeStruct((bb, s * kk, cf), F32),
        interpret=_INTERPRET,
    )(feat, ptsT, newp)


def _norm_relu(x, s, gb, rr):
    mean = s[0:1] * (1.0 / rr)
    ex2 = s[1:2] * (1.0 / rr)
    var = ex2 - mean * mean
    inv = 1.0 / jnp.sqrt(var + 1e-5)
    return jnp.maximum((x - mean) * inv * gb[0:1] + gb[1:2], 0.0)


def _layer_body(rr, first, *refs):
    if first:
        x_ref, wt_ref, b_ref, out_ref, acc_ref = refs
        x = x_ref[...]
    else:
        x_ref, wt_ref, b_ref, stats_ref, gb_ref, out_ref, acc_ref = refs
        x = _norm_relu(x_ref[...], stats_ref[...], gb_ref[...], rr)
    y = _dot(x, wt_ref[...]) + b_ref[...]
    out_ref[...] = y

    @pl.when(pl.program_id(0) == 0)
    def _():
        acc_ref[...] = jnp.zeros_like(acc_ref)

    s1 = jnp.sum(y, axis=0, keepdims=True)
    s2 = jnp.sum(y * y, axis=0, keepdims=True)
    acc_ref[...] += jnp.concatenate([s1, s2], axis=0)


def _layer(x, stats, gb, wt, b, rb):
    rr, cin = x.shape
    cout = wt.shape[1]
    first = stats is None
    grid = (rr // rb,)
    in_specs = [
        pl.BlockSpec((rb, cin), lambda i: (i, 0)),
        pl.BlockSpec((cin, cout), lambda i: (0, 0)),
        pl.BlockSpec((1, cout), lambda i: (0, 0)),
    ]
    args = [x, wt, b]
    if not first:
        in_specs += [pl.BlockSpec((2, cin), lambda i: (0, 0)),
                     pl.BlockSpec((2, cin), lambda i: (0, 0))]
        args += [stats, gb]
    y, acc = pl.pallas_call(
        functools.partial(_layer_body, float(rr), first),
        grid=grid,
        in_specs=in_specs,
        out_specs=[pl.BlockSpec((rb, cout), lambda i: (i, 0)),
                   pl.BlockSpec((2, cout), lambda i: (0, 0))],
        out_shape=[jax.ShapeDtypeStruct((rr, cout), F32),
                   jax.ShapeDtypeStruct((2, cout), F32)],
        interpret=_INTERPRET,
    )(*args)
    return y, acc


def _finish_body(rr, kk, rb, cc, x_ref, stats_ref, gb_ref, out_ref):
    x = x_ref[...]
    if kk > 1:
        x = jnp.max(x.reshape(rb // kk, kk, cc), axis=1)
    out_ref[...] = _norm_relu(x, stats_ref[...], gb_ref[...], rr)


def _finish(x, stats, gb, kk, rb):
    rr, cc = x.shape
    return pl.pallas_call(
        functools.partial(_finish_body, float(rr), kk, rb, cc),
        grid=(rr // rb,),
        in_specs=[
            pl.BlockSpec((rb, cc), lambda i: (i, 0)),
            pl.BlockSpec((2, cc), lambda i: (0, 0)),
            pl.BlockSpec((2, cc), lambda i: (0, 0)),
        ],
        out_specs=pl.BlockSpec((rb // kk, cc), lambda i: (i, 0)),
        out_shape=jax.ShapeDtypeStruct((rr // kk, cc), F32),
        interpret=_INTERPRET,
    )(x, stats, gb)


def _mlp_rows(x, layers, kk):
    rr = x.shape[0]
    rb = min(rr, 2048)
    stats = None
    gb = None
    for lyr in layers:
        wt = jnp.transpose(lyr['W'], (1, 0))
        b = lyr['b'][None, :]
        x, stats = _layer(x, stats, gb, wt, b, rb)
        gb = jnp.stack([lyr['gamma'], lyr['beta']])
    fin_rb = rb if rb % kk == 0 and rb >= kk else rr
    return _finish(x, stats, gb, kk, fin_rb)


def _interp_body(s2, c1, nb, x1_ref, x2T_ref, p2_ref, *rest):
    if c1 > 0:
        p1_ref, out_ref = rest
    else:
        (out_ref,) = rest
    x1 = x1_ref[0]
    x2T = x2T_ref[0]
    n1 = (x1[:, 0:1] * x1[:, 0:1] + x1[:, 1:2] * x1[:, 1:2]) + x1[:, 2:3] * x1[:, 2:3]
    n2 = (x2T[0:1] * x2T[0:1] + x2T[1:2] * x2T[1:2]) + x2T[2:3] * x2T[2:3]
    d = (-2.0 * _dot(x1, x2T) + n1) + n2
    lanes = lax.broadcasted_iota(jnp.int32, (nb, s2), 1)
    wmat = jnp.zeros((nb, s2), F32)
    wsum = jnp.zeros((nb, 1), F32)
    for _ in range(3):
        mv = jnp.min(d, axis=1, keepdims=True)
        idx = jnp.min(jnp.where(d == mv, lanes, s2), axis=1, keepdims=True)
        oh = lanes == idx
        w = 1.0 / (mv + 1e-8)
        wmat = wmat + jnp.where(oh, w, 0.0)
        wsum = wsum + w
        d = jnp.where(oh, 1e30, d)
    wmat = wmat / wsum
    interp = _dot(wmat, p2_ref[0])
    if c1 > 0:
        out_ref[0, :, :c1] = p1_ref[0]
        out_ref[0, :, c1:] = interp
    else:
        out_ref[0] = interp


def _interp(x1p, x2T, p2, p1, nb):
    bb, n1, _ = x1p.shape
    s2 = x2T.shape[2]
    c2 = p2.shape[2]
    c1 = 0 if p1 is None else p1.shape[2]
    in_specs = [
        pl.BlockSpec((1, nb, 3), lambda b, i: (b, i, 0)),
        pl.BlockSpec((1, 3, s2), lambda b, i: (b, 0, 0)),
        pl.BlockSpec((1, s2, c2), lambda b, i: (b, 0, 0)),
    ]
    args = [x1p, x2T, p2]
    if c1 > 0:
        in_specs.append(pl.BlockSpec((1, nb, c1), lambda b, i: (b, i, 0)))
        args.append(p1)
    return pl.pallas_call(
        functools.partial(_interp_body, s2, c1, nb),
        grid=(bb, n1 // nb),
        in_specs=in_specs,
        out_specs=pl.BlockSpec((1, nb, c1 + c2), lambda b, i: (b, i, 0)),
        out_shape=jax.ShapeDtypeStruct((bb, n1, c1 + c2), F32),
        interpret=_INTERPRET,
    )(*args)


SA_LEVELS = [
    (1024, [0.05, 0.1], [16, 32], 32),
    (256, [0.1, 0.2], [16, 32], 64),
    (64, [0.2, 0.4], [16, 32], 64),
]


def kernel(xyz, params):
    bb = xyz.shape[0]
    l0T = xyz
    l0p = jnp.transpose(xyz, (0, 2, 1))

    ptsT, pts, points = l0T, l0p, None
    level_data = []
    for li, (s, radii, ks, sbb) in enumerate(SA_LEVELS):
        name = 'sa%d' % (li + 1)
        newT = _fps(ptsT, s)
        newp = jnp.transpose(newT, (0, 2, 1))
        if points is None:
            feat = pts
        else:
            feat = jnp.concatenate([points, pts], axis=2)
        outs = []
        for radius, kk, layers in zip(radii, ks, params[name]):
            g = _group(feat, ptsT, newp, radius, kk, sbb)
            rows = g.reshape(bb * s * kk, feat.shape[2])
            outs.append(_mlp_rows(rows, layers, kk))
        new_points = jnp.concatenate(outs, axis=1).reshape(bb, s, -1)
        level_data.append((newT, newp, new_points))
        ptsT, pts, points = newT, newp, new_points

    l1T, l1p, l1_points = level_data[0]
    l2T, l2p, l2_points = level_data[1]
    l3T, l3p, l3_points = level_data[2]

    rows4 = jnp.concatenate([l3p.reshape(bb * 64, 3),
                             l3_points.reshape(bb * 64, -1)], axis=1)
    l4rows = _mlp_rows(rows4, params['sa4'], 64)
    global_features = l4rows.reshape(bb, 512)

    interp4 = jnp.broadcast_to(l4rows[:, None, :], (bb, 64, 512))
    np4 = jnp.concatenate([l3_points, interp4], axis=2).reshape(bb * 64, 1024)
    l3f = _mlp_rows(np4, params['fp4'], 1).reshape(bb, 64, 256)

    np3 = _interp(l2p, l3T, l3f, l2_points, 256)
    l2f = _mlp_rows(np3.reshape(bb * 256, 512), params['fp3'], 1).reshape(bb, 256, 256)

    np2 = _interp(l1p, l2T, l2f, l1_points, 512)
    l1f = _mlp_rows(np2.reshape(bb * 1024, 352), params['fp2'], 1).reshape(bb, 1024, 128)

    np1 = _interp(l0p, l1T, l1f, None, 512)
    l0f = _mlp_rows(np1.reshape(bb * 4096, 128), params['fp1'], 1)
    point_features = l0f.reshape(bb, 4096, 128)

    return (global_features, point_features)


# speedup vs baseline: 12.0590x; 1.0342x over previous
import functools

import jax
import jax.numpy as jnp
from jax import lax
from jax.experimental import pallas as pl

F32 = jnp.float32
HI = lax.Precision.HIGHEST
DEF = lax.Precision.DEFAULT
_INTERPRET = False


def _dot(a, b, prec=DEF):
    return lax.dot_general(a, b, (((1,), (0,)), ((), ())),
                           precision=prec, preferred_element_type=F32)


def _fps_body(npoint, n, xt_ref, out_ref):
    x = xt_ref[:, 0, :]
    y = xt_ref[:, 1, :]
    z = xt_ref[:, 2, :]
    bb = x.shape[0]
    lanes = lax.broadcasted_iota(jnp.int32, (bb, n), 1)
    slot = lax.broadcasted_iota(jnp.int32, (bb, npoint), 1)

    def body(i, carry):
        dist, far, cx, cy, cz = carry
        m = lanes == far
        c0 = jnp.sum(jnp.where(m, x, 0.0), axis=1, keepdims=True)
        c1 = jnp.sum(jnp.where(m, y, 0.0), axis=1, keepdims=True)
        c2 = jnp.sum(jnp.where(m, z, 0.0), axis=1, keepdims=True)
        hit = slot == i
        cx = jnp.where(hit, c0, cx)
        cy = jnp.where(hit, c1, cy)
        cz = jnp.where(hit, c2, cz)
        dx = x - c0
        dy = y - c1
        dz = z - c2
        dn = (dx * dx + dy * dy) + dz * dz
        dist = jnp.minimum(dist, dn)
        mv = jnp.max(dist, axis=1, keepdims=True)
        far = jnp.min(jnp.where(dist == mv, lanes, n), axis=1, keepdims=True)
        return dist, far, cx, cy, cz

    zc = jnp.zeros((bb, npoint), F32)
    _, _, cx, cy, cz = lax.fori_loop(
        0, npoint, body,
        (jnp.full((bb, n), 1e10, F32), jnp.zeros((bb, 1), jnp.int32),
         zc, zc, zc))
    out_ref[:, 0:1, :] = cx[:, None, :]
    out_ref[:, 1:2, :] = cy[:, None, :]
    out_ref[:, 2:3, :] = cz[:, None, :]


def _fps(xt, npoint):
    bb, _, n = xt.shape
    return pl.pallas_call(
        functools.partial(_fps_body, npoint, n),
        out_shape=jax.ShapeDtypeStruct((bb, 3, npoint), F32),
        interpret=_INTERPRET,
    )(xt)


def _cumsum_lanes(c, n):
    sh = 1
    while sh < n:
        pad = jnp.zeros((c.shape[0], sh), jnp.int32)
        c = c + jnp.concatenate([pad, c[:, :n - sh]], axis=1)
        sh *= 2
    return c


def _proj_body(x_ref, w_ref, out_ref):
    out_ref[0] = _dot(x_ref[0], w_ref[...])


def _proj(feat, wt):
    bb, n, cf = feat.shape
    c1 = wt.shape[1]
    return pl.pallas_call(
        _proj_body,
        grid=(bb,),
        in_specs=[pl.BlockSpec((1, n, cf), lambda b: (b, 0, 0)),
                  pl.BlockSpec((cf, c1), lambda b: (0, 0))],
        out_specs=pl.BlockSpec((1, n, c1), lambda b: (b, 0, 0)),
        out_shape=jax.ShapeDtypeStruct((bb, n, c1), F32),
        interpret=_INTERPRET,
    )(feat, wt)


def _group_body(r2, kk, n, sb, proj_ref, ptsT_ref, newp_ref, wx_ref, b_ref,
                out_ref, acc_ref):
    ptsT = ptsT_ref[0]
    newp = newp_ref[0]
    rn = (ptsT[0:1] * ptsT[0:1] + ptsT[1:2] * ptsT[1:2]) + ptsT[2:3] * ptsT[2:3]
    s0 = newp[:, 0:1]
    s1 = newp[:, 1:2]
    s2 = newp[:, 2:3]
    sn = (s0 * s0 + s1 * s1) + s2 * s2
    d = (-2.0 * _dot(newp, ptsT) + sn) + rn
    mask = (d <= r2).astype(jnp.int32)
    c = _cumsum_lanes(mask, n)
    total = c[:, n - 1:n]
    cm = c * mask
    kv = lax.broadcasted_iota(jnp.int32, (1, kk), 1) + 1
    target = jnp.where(kv <= total, kv, 1)
    oh = (cm[:, None, :] == target[:, :, None]).reshape(sb * kk, n).astype(F32)
    c1 = proj_ref.shape[2]
    corr = b_ref[...] - _dot(newp, wx_ref[...])
    corr = jnp.broadcast_to(corr[:, None, :], (sb, kk, c1)).reshape(sb * kk, c1)
    y = _dot(oh, proj_ref[0], DEF) + corr
    out_ref[0] = y

    @pl.when((pl.program_id(0) == 0) & (pl.program_id(1) == 0))
    def _():
        acc_ref[...] = jnp.zeros_like(acc_ref)

    sm1 = jnp.sum(y, axis=0, keepdims=True)
    sm2 = jnp.sum(y * y, axis=0, keepdims=True)
    acc_ref[...] += jnp.concatenate([sm1, sm2], axis=0)


def _group(proj, ptsT, newp, wx, b1, radius, kk, sbb):
    bb, n, c1 = proj.shape
    s = newp.shape[1]
    return pl.pallas_call(
        functools.partial(_group_body, radius * radius, kk, n, sbb),
        grid=(bb, s // sbb),
        in_specs=[
            pl.BlockSpec((1, n, c1), lambda b, i: (b, 0, 0)),
            pl.BlockSpec((1, 3, n), lambda b, i: (b, 0, 0)),
            pl.BlockSpec((1, sbb, 3), lambda b, i: (b, i, 0)),
            pl.BlockSpec((3, c1), lambda b, i: (0, 0)),
            pl.BlockSpec((1, c1), lambda b, i: (0, 0)),
        ],
        out_specs=[pl.BlockSpec((1, sbb * kk, c1), lambda b, i: (b, i, 0)),
                   pl.BlockSpec((2, c1), lambda b, i: (0, 0))],
        out_shape=[jax.ShapeDtypeStruct((bb, s * kk, c1), F32),
                   jax.ShapeDtypeStruct((2, c1), F32)],
        interpret=_INTERPRET,
    )(proj, ptsT, newp, wx, b1)


def _norm_relu(x, s, gb, rr):
    mean = s[0:1] * (1.0 / rr)
    ex2 = s[1:2] * (1.0 / rr)
    var = ex2 - mean * mean
    inv = 1.0 / jnp.sqrt(var + 1e-5)
    return jnp.maximum((x - mean) * inv * gb[0:1] + gb[1:2], 0.0)


def _layer_body(rr, first, *refs):
    if first:
        x_ref, wt_ref, b_ref, out_ref, acc_ref = refs
        x = x_ref[...]
    else:
        x_ref, wt_ref, b_ref, stats_ref, gb_ref, out_ref, acc_ref = refs
        x = _norm_relu(x_ref[...], stats_ref[...], gb_ref[...], rr)
    y = _dot(x, wt_ref[...]) + b_ref[...]
    out_ref[...] = y

    @pl.when(pl.program_id(0) == 0)
    def _():
        acc_ref[...] = jnp.zeros_like(acc_ref)

    s1 = jnp.sum(y, axis=0, keepdims=True)
    s2 = jnp.sum(y * y, axis=0, keepdims=True)
    acc_ref[...] += jnp.concatenate([s1, s2], axis=0)


def _layer(x, stats, gb, wt, b, rb):
    rr, cin = x.shape
    cout = wt.shape[1]
    first = stats is None
    grid = (rr // rb,)
    in_specs = [
        pl.BlockSpec((rb, cin), lambda i: (i, 0)),
        pl.BlockSpec((cin, cout), lambda i: (0, 0)),
        pl.BlockSpec((1, cout), lambda i: (0, 0)),
    ]
    args = [x, wt, b]
    if not first:
        in_specs += [pl.BlockSpec((2, cin), lambda i: (0, 0)),
                     pl.BlockSpec((2, cin), lambda i: (0, 0))]
        args += [stats, gb]
    y, acc = pl.pallas_call(
        functools.partial(_layer_body, float(rr), first),
        grid=grid,
        in_specs=in_specs,
        out_specs=[pl.BlockSpec((rb, cout), lambda i: (i, 0)),
                   pl.BlockSpec((2, cout), lambda i: (0, 0))],
        out_shape=[jax.ShapeDtypeStruct((rr, cout), F32),
                   jax.ShapeDtypeStruct((2, cout), F32)],
        interpret=_INTERPRET,
    )(*args)
    return y, acc


def _finish_body(rr, kk, rb, cc, x_ref, stats_ref, gb_ref, out_ref):
    x = x_ref[...]
    if kk > 1:
        x = jnp.max(x.reshape(rb // kk, kk, cc), axis=1)
    out_ref[...] = _norm_relu(x, stats_ref[...], gb_ref[...], rr)


def _finish(x, stats, gb, kk, rb):
    rr, cc = x.shape
    return pl.pallas_call(
        functools.partial(_finish_body, float(rr), kk, rb, cc),
        grid=(rr // rb,),
        in_specs=[
            pl.BlockSpec((rb, cc), lambda i: (i, 0)),
            pl.BlockSpec((2, cc), lambda i: (0, 0)),
            pl.BlockSpec((2, cc), lambda i: (0, 0)),
        ],
        out_specs=pl.BlockSpec((rb // kk, cc), lambda i: (i, 0)),
        out_shape=jax.ShapeDtypeStruct((rr // kk, cc), F32),
        interpret=_INTERPRET,
    )(x, stats, gb)


def _mlp_rows(x, layers, kk, stats=None, gb=None):
    rr = x.shape[0]
    rb = min(rr, 2048)
    for lyr in layers:
        wt = jnp.transpose(lyr['W'], (1, 0))
        b = lyr['b'][None, :]
        x, stats = _layer(x, stats, gb, wt, b, rb)
        gb = jnp.stack([lyr['gamma'], lyr['beta']])
    fin_rb = rb if rb % kk == 0 and rb >= kk else rr
    return _finish(x, stats, gb, kk, fin_rb)


def _interp_body(s2, c1, nb, x1_ref, x2T_ref, p2_ref, *rest):
    if c1 > 0:
        p1_ref, out_ref = rest
    else:
        (out_ref,) = rest
    x1 = x1_ref[0]
    x2T = x2T_ref[0]
    n1 = (x1[:, 0:1] * x1[:, 0:1] + x1[:, 1:2] * x1[:, 1:2]) + x1[:, 2:3] * x1[:, 2:3]
    n2 = (x2T[0:1] * x2T[0:1] + x2T[1:2] * x2T[1:2]) + x2T[2:3] * x2T[2:3]
    d = (-2.0 * _dot(x1, x2T) + n1) + n2
    lanes = lax.broadcasted_iota(jnp.int32, (nb, s2), 1)
    wmat = jnp.zeros((nb, s2), F32)
    wsum = jnp.zeros((nb, 1), F32)
    for _ in range(3):
        mv = jnp.min(d, axis=1, keepdims=True)
        idx = jnp.min(jnp.where(d == mv, lanes, s2), axis=1, keepdims=True)
        oh = lanes == idx
        w = 1.0 / (mv + 1e-8)
        wmat = wmat + jnp.where(oh, w, 0.0)
        wsum = wsum + w
        d = jnp.where(oh, 1e30, d)
    wmat = wmat / wsum
    interp = _dot(wmat, p2_ref[0])
    if c1 > 0:
        out_ref[0, :, :c1] = p1_ref[0]
        out_ref[0, :, c1:] = interp
    else:
        out_ref[0] = interp


def _interp(x1p, x2T, p2, p1, nb):
    bb, n1, _ = x1p.shape
    s2 = x2T.shape[2]
    c2 = p2.shape[2]
    c1 = 0 if p1 is None else p1.shape[2]
    in_specs = [
        pl.BlockSpec((1, nb, 3), lambda b, i: (b, i, 0)),
        pl.BlockSpec((1, 3, s2), lambda b, i: (b, 0, 0)),
        pl.BlockSpec((1, s2, c2), lambda b, i: (b, 0, 0)),
    ]
    args = [x1p, x2T, p2]
    if c1 > 0:
        in_specs.append(pl.BlockSpec((1, nb, c1), lambda b, i: (b, i, 0)))
        args.append(p1)
    return pl.pallas_call(
        functools.partial(_interp_body, s2, c1, nb),
        grid=(bb, n1 // nb),
        in_specs=in_specs,
        out_specs=pl.BlockSpec((1, nb, c1 + c2), lambda b, i: (b, i, 0)),
        out_shape=jax.ShapeDtypeStruct((bb, n1, c1 + c2), F32),
        interpret=_INTERPRET,
    )(*args)


SA_LEVELS = [
    (1024, [0.05, 0.1], [16, 32], 32),
    (256, [0.1, 0.2], [16, 32], 64),
    (64, [0.2, 0.4], [16, 32], 64),
]


def kernel(xyz, params):
    bb = xyz.shape[0]
    l0T = xyz
    l0p = jnp.transpose(xyz, (0, 2, 1))

    ptsT, pts, points = l0T, l0p, None
    level_data = []
    for li, (s, radii, ks, sbb) in enumerate(SA_LEVELS):
        name = 'sa%d' % (li + 1)
        newT = _fps(ptsT, s)
        newp = jnp.transpose(newT, (0, 2, 1))
        if points is None:
            feat = pts
        else:
            feat = jnp.concatenate([points, pts], axis=2)
        outs = []
        for radius, kk, layers in zip(radii, ks, params[name]):
            l1 = layers[0]
            wt1 = jnp.transpose(l1['W'], (1, 0))
            proj = _proj(feat, wt1)
            wx = wt1[-3:, :]
            g, acc = _group(proj, ptsT, newp, wx, l1['b'][None, :],
                            radius, kk, sbb)
            rows = g.reshape(bb * s * kk, g.shape[2])
            gb1 = jnp.stack([l1['gamma'], l1['beta']])
            outs.append(_mlp_rows(rows, layers[1:], kk, stats=acc, gb=gb1))
        new_points = jnp.concatenate(outs, axis=1).reshape(bb, s, -1)
        level_data.append((newT, newp, new_points))
        ptsT, pts, points = newT, newp, new_points

    l1T, l1p, l1_points = level_data[0]
    l2T, l2p, l2_points = level_data[1]
    l3T, l3p, l3_points = level_data[2]

    rows4 = jnp.concatenate([l3p.reshape(bb * 64, 3),
                             l3_points.reshape(bb * 64, -1)], axis=1)
    l4rows = _mlp_rows(rows4, params['sa4'], 64)
    global_features = l4rows.reshape(bb, 512)

    interp4 = jnp.broadcast_to(l4rows[:, None, :], (bb, 64, 512))
    np4 = jnp.concatenate([l3_points, interp4], axis=2).reshape(bb * 64, 1024)
    l3f = _mlp_rows(np4, params['fp4'], 1).reshape(bb, 64, 256)

    np3 = _interp(l2p, l3T, l3f, l2_points, 256)
    l2f = _mlp_rows(np3.reshape(bb * 256, 512), params['fp3'], 1).reshape(bb, 256, 256)

    np2 = _interp(l1p, l2T, l2f, l1_points, 512)
    l1f = _mlp_rows(np2.reshape(bb * 1024, 352), params['fp2'], 1).reshape(bb, 1024, 128)

    np1 = _interp(l0p, l1T, l1f, None, 512)
    l0f = _mlp_rows(np1.reshape(bb * 4096, 128), params['fp1'], 1)
    point_features = l0f.reshape(bb, 4096, 128)

    return (global_features, point_features)
